# Initial kernel scaffold; baseline (speedup 1.0000x reference)
import jax
import jax.numpy as jnp
from jax.experimental import pallas as pl


def kernel(x, edge_index, W1, a1_src, a1_dst, b1, W2, a2_src, a2_dst, b2):
    raise NotImplementedError("write your pallas kernel here")


# speedup vs baseline: 4.7542x; 4.7542x over previous
import functools

import jax
import jax.numpy as jnp
from jax import lax
from jax.experimental import pallas as pl
from jax.experimental.pallas import tpu as pltpu
from jax.experimental.pallas import tpu_sc as plsc

N = 10000
E = 320000
D_IN = 128
HID = 64
HEADS = 8
D_OUT = 128

N_PAD = 10240
E_PAD = 327680
NC, NS, LANES = 2, 16, 16
KBLK = 1024
SUB = KBLK // 128
GROW = 1024
ROWS_PER_TILE = N_PAD // NS

_mesh = plsc.VectorSubcoreMesh(
    core_axis_name="c", subcore_axis_name="s", num_cores=NC, num_subcores=NS)
_sc_params = pltpu.CompilerParams(
    needs_layout_passes=False, use_tc_tiling_on_sc=False)


def _mm1_body(x_ref, w_ref, asr_ref, adr_ref, h_ref, s_ref, d_ref):
    h = jnp.dot(x_ref[...], w_ref[...], preferred_element_type=jnp.float32)
    s_ref[...] = jnp.dot(h, asr_ref[...], preferred_element_type=jnp.float32)
    d_ref[...] = jnp.dot(h, adr_ref[...], preferred_element_type=jnp.float32)
    for j in range(8):
        h_ref[j] = h[:, j * 64:(j + 1) * 64]


def _mm1(x_pad, W1, Asrc, Adst):
    BR = 128
    return pl.pallas_call(
        _mm1_body,
        grid=(N_PAD // BR,),
        in_specs=[
            pl.BlockSpec((BR, D_IN), lambda i: (i, 0)),
            pl.BlockSpec((D_IN, HEADS * HID), lambda i: (0, 0)),
            pl.BlockSpec((HEADS * HID, HEADS), lambda i: (0, 0)),
            pl.BlockSpec((HEADS * HID, HEADS), lambda i: (0, 0)),
        ],
        out_specs=[
            pl.BlockSpec((8, BR, 64), lambda i: (0, i, 0)),
            pl.BlockSpec((BR, HEADS), lambda i: (i, 0)),
            pl.BlockSpec((BR, HEADS), lambda i: (i, 0)),
        ],
        out_shape=[
            jax.ShapeDtypeStruct((8, N_PAD, 64), jnp.float32),
            jax.ShapeDtypeStruct((N_PAD, HEADS), jnp.float32),
            jax.ShapeDtypeStruct((N_PAD, HEADS), jnp.float32),
        ],
    )(x_pad, W1, Asrc, Adst)


def _rd_body(p0_ref, p1_ref, out_ref):
    den = p0_ref[...] + p1_ref[...]
    out_ref[...] = jnp.transpose(1.0 / (den + 1e-16))


def _rdenom(partials, H):
    BR = 128
    p = partials.reshape(2, N_PAD, H)
    return pl.pallas_call(
        _rd_body,
        grid=(N_PAD // BR,),
        in_specs=[
            pl.BlockSpec((BR, H), lambda i: (i, 0)),
            pl.BlockSpec((BR, H), lambda i: (i, 0)),
        ],
        out_specs=pl.BlockSpec((H, BR), lambda i: (0, i)),
        out_shape=jax.ShapeDtypeStruct((H, N_PAD), jnp.float32),
    )(p[0], p[1])


def _mm2_body(o0, o1, o2, o3, o4, o5, o6, o7, b1_ref, w2_ref, a2s_ref,
              a2d_ref, h2_ref, ls_ref, ld_ref):
    i = pl.program_id(0)
    hcat = jnp.concatenate(
        [o0[...], o1[...], o2[...], o3[...],
         o4[...], o5[...], o6[...], o7[...]], axis=1)
    h = jnp.maximum(hcat + b1_ref[...], 0.0)
    rows = i * h.shape[0] + lax.broadcasted_iota(jnp.int32, (h.shape[0], 1), 0)
    h = jnp.where(rows < N, h, 0.0)
    h2 = jnp.dot(h, w2_ref[...], preferred_element_type=jnp.float32)
    pad7 = jnp.zeros((h2.shape[0], 7), jnp.float32)
    ls_ref[...] = jnp.concatenate(
        [jnp.sum(h2 * a2s_ref[...], axis=1, keepdims=True), pad7], axis=1)
    ld_ref[...] = jnp.concatenate(
        [jnp.sum(h2 * a2d_ref[...], axis=1, keepdims=True), pad7], axis=1)
    h2_ref[0] = h2[:, :64]
    h2_ref[1] = h2[:, 64:]


def _mm2(hagg, b1, W2, a2_src, a2_dst):
    BR = 128
    o = hagg.reshape(8, N_PAD, 64)
    return pl.pallas_call(
        _mm2_body,
        grid=(N_PAD // BR,),
        in_specs=[pl.BlockSpec((BR, 64), lambda i: (i, 0))] * 8 + [
            pl.BlockSpec((1, HEADS * HID), lambda i: (0, 0)),
            pl.BlockSpec((HEADS * HID, D_OUT), lambda i: (0, 0)),
            pl.BlockSpec((1, D_OUT), lambda i: (0, 0)),
            pl.BlockSpec((1, D_OUT), lambda i: (0, 0)),
        ],
        out_specs=[
            pl.BlockSpec((2, BR, 64), lambda i: (0, i, 0)),
            pl.BlockSpec((BR, HEADS), lambda i: (i, 0)),
            pl.BlockSpec((BR, HEADS), lambda i: (i, 0)),
        ],
        out_shape=[
            jax.ShapeDtypeStruct((2, N_PAD, 64), jnp.float32),
            jax.ShapeDtypeStruct((N_PAD, HEADS), jnp.float32),
            jax.ShapeDtypeStruct((N_PAD, HEADS), jnp.float32),
        ],
    )(o[0], o[1], o[2], o[3], o[4], o[5], o[6], o[7], b1.reshape(1, -1), W2,
      a2_src.reshape(1, -1), a2_dst.reshape(1, -1))


def _out_body(q0, q1, b2_ref, out_ref):
    out_ref[...] = jnp.concatenate([q0[...], q1[...]], axis=1) + b2_ref[...]


def _combine_out(o2agg, b2):
    BR = 128
    q = o2agg.reshape(2, N_PAD, 64)
    return pl.pallas_call(
        _out_body,
        grid=(N_PAD // BR,),
        in_specs=[
            pl.BlockSpec((BR, 64), lambda i: (i, 0)),
            pl.BlockSpec((BR, 64), lambda i: (i, 0)),
            pl.BlockSpec((1, D_OUT), lambda i: (0, 0)),
        ],
        out_specs=pl.BlockSpec((BR, D_OUT), lambda i: (i, 0)),
        out_shape=jax.ShapeDtypeStruct((N_PAD, D_OUT), jnp.float32),
    )(q[0], q[1], b2.reshape(1, -1))


def _iota16():
    return lax.iota(jnp.int32, 16)


def _make_pass_a(H):
    EPT = E_PAD // (NC * NS)
    NBLK = EPT // KBLK
    NSL = KBLK * H // LANES

    scratch = [
        pltpu.VMEM((KBLK,), jnp.int32),
        pltpu.VMEM((SUB, 128), jnp.int32),
        pltpu.VMEM((KBLK, H), jnp.float32),
        pltpu.VMEM((KBLK, H), jnp.float32),
        pltpu.VMEM((KBLK, H), jnp.float32),
        pltpu.VMEM((H, KBLK), jnp.float32),
        pltpu.VMEM_SHARED((N_PAD, H), jnp.float32),
        pltpu.SemaphoreType.DMA,
    ]

    @functools.partial(
        pl.kernel, mesh=_mesh, compiler_params=_sc_params,
        out_type=(
            jax.ShapeDtypeStruct((2 * N_PAD, H), jnp.float32),
            jax.ShapeDtypeStruct((H * E_PAD,), jnp.float32),
        ),
        scratch_types=scratch,
    )
    def pass_a(src_hbm, dst2d_hbm, asrc_hbm, adst_hbm, zeros_hbm,
               den_out, ext_out,
               src_v, dst_v, asr_v, adr_v, exr_v, ext_v, den_sh, sem):
        c = lax.axis_index("c")
        s = lax.axis_index("s")
        wid = s * NC + c
        base_edges = wid * EPT

        pltpu.sync_copy(zeros_hbm.at[pl.ds(s * ROWS_PER_TILE, ROWS_PER_TILE)],
                        den_sh.at[pl.ds(s * ROWS_PER_TILE, ROWS_PER_TILE)])
        plsc.subcore_barrier()

        def blk(b, carry):
            ebase = pl.multiple_of(base_edges + b * KBLK, KBLK)
            rbase = pl.multiple_of(ebase // 128, SUB)
            pltpu.sync_copy(src_hbm.at[pl.ds(ebase, KBLK)], src_v)
            pltpu.sync_copy(dst2d_hbm.at[pl.ds(rbase, SUB)], dst_v)
            cps = []
            for j in range(SUB):
                cps.append(pltpu.async_copy(
                    asrc_hbm.at[src_v.at[pl.ds(j * 128, 128)]],
                    asr_v.at[pl.ds(j * 128, 128)], sem))
                cps.append(pltpu.async_copy(
                    adst_hbm.at[dst_v.at[j]],
                    adr_v.at[pl.ds(j * 128, 128)], sem))
            for cp in cps:
                cp.wait()

            SH = H.bit_length() - 1

            def comp(v, c2):
                flat = v * LANES + _iota16()
                r = lax.shift_right_logical(flat, SH)
                col = lax.bitwise_and(flat, H - 1)
                e = (plsc.load_gather(asr_v, [r, col])
                     + plsc.load_gather(adr_v, [r, col]))
                e = jnp.where(e >= 0.0, e, 0.2 * e)
                ex = jnp.exp(e)
                plsc.store_scatter(exr_v, [r, col], ex)
                plsc.store_scatter(ext_v, [col, r], ex)
                return c2
            lax.fori_loop(0, NSL, comp, 0, unroll=4)

            for j in range(SUB):
                pltpu.sync_copy(exr_v.at[pl.ds(j * 128, 128)],
                                den_sh.at[dst_v.at[j]], add=True)
            for h in range(H):
                pltpu.sync_copy(
                    ext_v.at[h],
                    ext_out.at[pl.ds(
                        pl.multiple_of(h * E_PAD + ebase, KBLK), KBLK)])
            return carry
        lax.fori_loop(0, NBLK, blk, 0)

        plsc.subcore_barrier()
        pltpu.sync_copy(
            den_sh.at[pl.ds(s * ROWS_PER_TILE, ROWS_PER_TILE)],
            den_out.at[pl.ds(
                pl.multiple_of(c * N_PAD + s * ROWS_PER_TILE, ROWS_PER_TILE),
                ROWS_PER_TILE)])

    return pass_a


def _make_pass_b(NCHUNK, DC, HPC, NHEADS):
    EPT = E_PAD // NS
    NBLK = EPT // KBLK
    CPS = NCHUNK // NC
    DPH = DC // HPC

    scratch = [
        pltpu.VMEM((KBLK,), jnp.int32),
        pltpu.VMEM((SUB, 128), jnp.int32),
        pltpu.VMEM((GROW, DC), jnp.float32),
    ] + [pltpu.VMEM((KBLK,), jnp.float32) for _ in range(HPC)] \
      + [pltpu.VMEM((N_PAD,), jnp.float32) for _ in range(HPC)] + [
        pltpu.VMEM_SHARED((N_PAD, DC), jnp.float32),
        pltpu.SemaphoreType.DMA,
    ]

    @functools.partial(
        pl.kernel, mesh=_mesh, compiler_params=_sc_params,
        out_type=jax.ShapeDtypeStruct((NCHUNK * N_PAD, DC), jnp.float32),
        scratch_types=scratch,
    )
    def pass_b(src_hbm, dst2d_hbm, hblk_hbm, ext_hbm, rdt_hbm,
               zeros_hbm, out_hbm, *refs):
        idx_v = refs[0]
        dst_v = refs[1]
        rows_v = refs[2]
        ex_vs = [refs[3 + k] for k in range(HPC)]
        rd_vs = [refs[3 + HPC + k] for k in range(HPC)]
        acc_sh = refs[3 + 2 * HPC]
        sem = refs[4 + 2 * HPC]

        c = lax.axis_index("c")
        s = lax.axis_index("s")

        def chunk_body(jj, carry0):
            chunk = c * CPS + jj
            heads = [(chunk * NHEADS) // NCHUNK + hh for hh in range(HPC)]
            for hh in range(HPC):
                pltpu.sync_copy(
                    rdt_hbm.at[pl.ds(
                        pl.multiple_of(heads[hh] * N_PAD, N_PAD), N_PAD)],
                    rd_vs[hh])
            pltpu.sync_copy(
                zeros_hbm.at[pl.ds(s * ROWS_PER_TILE, ROWS_PER_TILE)],
                acc_sh.at[pl.ds(s * ROWS_PER_TILE, ROWS_PER_TILE)])
            plsc.subcore_barrier()

            off = chunk * N_PAD

            def blk(b, carry):
                ebase = pl.multiple_of(s * EPT + b * KBLK, KBLK)
                rbase = pl.multiple_of(ebase // 128, SUB)
                pltpu.sync_copy(src_hbm.at[pl.ds(ebase, KBLK)], idx_v)
                pltpu.sync_copy(dst2d_hbm.at[pl.ds(rbase, SUB)], dst_v)

                def addoff(v, c2):
                    sl = pl.ds(v * LANES, LANES)
                    idx_v[sl] = idx_v[sl] + off
                    return c2
                lax.fori_loop(0, KBLK // LANES, addoff, 0, unroll=4)

                for hh in range(HPC):
                    pltpu.sync_copy(
                        ext_hbm.at[pl.ds(
                            pl.multiple_of(heads[hh] * E_PAD + ebase, KBLK),
                            KBLK)],
                        ex_vs[hh])

                for half in range(KBLK // GROW):
                    hoff = half * GROW
                    cps = []
                    for j in range(GROW // 128):
                        cps.append(pltpu.async_copy(
                            hblk_hbm.at[idx_v.at[pl.ds(hoff + j * 128, 128)]],
                            rows_v.at[pl.ds(j * 128, 128)], sem))
                    for cp in cps:
                        cp.wait()

                    for g in range(GROW // LANES):
                        rowid = g * LANES + _iota16()
                        fo = hoff + g * LANES
                        d16 = dst_v.at[fo // 128][pl.ds(fo % 128, LANES)]
                        for hh in range(HPC):
                            exh = ex_vs[hh][pl.ds(hoff + g * LANES, LANES)]
                            al = exh * plsc.load_gather(rd_vs[hh], [d16])

                            def col(dd, c2, hh=hh, al=al, rowid=rowid):
                                d = (jnp.zeros((16,), jnp.int32)
                                     + (hh * DPH + dd))
                                cv = plsc.load_gather(rows_v, [rowid, d])
                                plsc.store_scatter(rows_v, [rowid, d], cv * al)
                                return c2
                            lax.fori_loop(0, DPH, col, 0, unroll=4)

                    for j in range(GROW // 128):
                        pltpu.sync_copy(
                            rows_v.at[pl.ds(j * 128, 128)],
                            acc_sh.at[dst_v.at[half * (GROW // 128) + j]],
                            add=True)
                return carry
            lax.fori_loop(0, NBLK, blk, 0)

            plsc.subcore_barrier()
            pltpu.sync_copy(
                acc_sh.at[pl.ds(s * ROWS_PER_TILE, ROWS_PER_TILE)],
                out_hbm.at[pl.ds(
                    pl.multiple_of(off + s * ROWS_PER_TILE, ROWS_PER_TILE),
                    ROWS_PER_TILE)])
            plsc.subcore_barrier()
            return carry0
        lax.fori_loop(0, CPS, chunk_body, 0)

    return pass_b


_pass_a_l1 = _make_pass_a(HEADS)
_pass_b_l1 = _make_pass_b(8, 64, 1, 8)
_pass_b_l2 = _make_pass_b(2, 64, 1, 1)


def kernel(x, edge_index, W1, a1_src, a1_dst, b1, W2, a2_src, a2_dst, b2):
    f32 = jnp.float32
    x_pad = jnp.pad(x, ((0, N_PAD - N), (0, 0)))
    src = edge_index[0]
    dst = edge_index[1]
    src_p = jnp.concatenate([src, jnp.zeros((E_PAD - E,), jnp.int32)])
    dst_p = jnp.concatenate([dst, jnp.full((E_PAD - E,), N_PAD - 1, jnp.int32)])
    dst2d = dst_p.reshape(E_PAD // 128, 128)

    eye = jnp.eye(HEADS, dtype=f32)
    Asrc = (a1_src[:, :, None] * eye[:, None, :]).reshape(HEADS * HID, HEADS)
    Adst = (a1_dst[:, :, None] * eye[:, None, :]).reshape(HEADS * HID, HEADS)

    z64 = jnp.zeros((N_PAD, 64), f32)
    z8 = jnp.zeros((N_PAD, HEADS), f32)

    hblk, asrc1, adst1 = _mm1(x_pad, W1, Asrc, Adst)
    den1, ext1 = _pass_a_l1(src_p, dst2d, asrc1, adst1, z8)
    rdt1 = _rdenom(den1, HEADS).reshape(-1)
    hagg = _pass_b_l1(src_p, dst2d, hblk.reshape(8 * N_PAD, 64), ext1,
                      rdt1, z64)

    h2blk, asrc2, adst2 = _mm2(hagg, b1, W2, a2_src, a2_dst)
    den2, ext2 = _pass_a_l1(src_p, dst2d, asrc2, adst2, z8)
    rdt2 = _rdenom(den2, HEADS).reshape(-1)
    o2agg = _pass_b_l2(src_p, dst2d, h2blk.reshape(2 * N_PAD, 64), ext2,
                       rdt2, z64)

    out = _combine_out(o2agg, b2)
    return out[:N]


# speedup vs baseline: 5.1864x; 1.0909x over previous
import functools

import jax
import jax.numpy as jnp
from jax import lax
from jax.experimental import pallas as pl
from jax.experimental.pallas import tpu as pltpu
from jax.experimental.pallas import tpu_sc as plsc

N = 10000
E = 320000
D_IN = 128
HID = 64
HEADS = 8
D_OUT = 128

N_PAD = 10240
E_PAD = 327680
NC, NS, LANES = 2, 16, 16
KBLK = 1024
SUB = KBLK // 128
GROW = 512
ROWS_PER_TILE = N_PAD // NS

_mesh = plsc.VectorSubcoreMesh(
    core_axis_name="c", subcore_axis_name="s", num_cores=NC, num_subcores=NS)
_sc_params = pltpu.CompilerParams(
    needs_layout_passes=False, use_tc_tiling_on_sc=False)


def _mm1_body(x_ref, w_ref, asr_ref, adr_ref, h_ref, s_ref, d_ref):
    h = jnp.dot(x_ref[...], w_ref[...], preferred_element_type=jnp.float32)
    s_ref[...] = jnp.dot(h, asr_ref[...], preferred_element_type=jnp.float32)
    d_ref[...] = jnp.dot(h, adr_ref[...], preferred_element_type=jnp.float32)
    for j in range(8):
        h_ref[j] = h[:, j * 64:(j + 1) * 64]


def _mm1(x_pad, W1, Asrc, Adst):
    BR = 128
    return pl.pallas_call(
        _mm1_body,
        grid=(N_PAD // BR,),
        in_specs=[
            pl.BlockSpec((BR, D_IN), lambda i: (i, 0)),
            pl.BlockSpec((D_IN, HEADS * HID), lambda i: (0, 0)),
            pl.BlockSpec((HEADS * HID, HEADS), lambda i: (0, 0)),
            pl.BlockSpec((HEADS * HID, HEADS), lambda i: (0, 0)),
        ],
        out_specs=[
            pl.BlockSpec((8, BR, 64), lambda i: (0, i, 0)),
            pl.BlockSpec((BR, HEADS), lambda i: (i, 0)),
            pl.BlockSpec((BR, HEADS), lambda i: (i, 0)),
        ],
        out_shape=[
            jax.ShapeDtypeStruct((8, N_PAD, 64), jnp.float32),
            jax.ShapeDtypeStruct((N_PAD, HEADS), jnp.float32),
            jax.ShapeDtypeStruct((N_PAD, HEADS), jnp.float32),
        ],
    )(x_pad, W1, Asrc, Adst)


def _rd_body(p0_ref, p1_ref, out_ref):
    den = p0_ref[...] + p1_ref[...]
    out_ref[...] = jnp.transpose(1.0 / (den + 1e-16))


def _rdenom(partials, H):
    BR = 128
    p = partials.reshape(2, N_PAD, H)
    return pl.pallas_call(
        _rd_body,
        grid=(N_PAD // BR,),
        in_specs=[
            pl.BlockSpec((BR, H), lambda i: (i, 0)),
            pl.BlockSpec((BR, H), lambda i: (i, 0)),
        ],
        out_specs=pl.BlockSpec((H, BR), lambda i: (0, i)),
        out_shape=jax.ShapeDtypeStruct((H, N_PAD), jnp.float32),
    )(p[0], p[1])


def _mm2_body(o0, o1, o2, o3, o4, o5, o6, o7, b1_ref, w2_ref, a2s_ref,
              a2d_ref, h2_ref, ls_ref, ld_ref):
    i = pl.program_id(0)
    hcat = jnp.concatenate(
        [o0[...], o1[...], o2[...], o3[...],
         o4[...], o5[...], o6[...], o7[...]], axis=1)
    h = jnp.maximum(hcat + b1_ref[...], 0.0)
    rows = i * h.shape[0] + lax.broadcasted_iota(jnp.int32, (h.shape[0], 1), 0)
    h = jnp.where(rows < N, h, 0.0)
    h2 = jnp.dot(h, w2_ref[...], preferred_element_type=jnp.float32)
    pad7 = jnp.zeros((h2.shape[0], 7), jnp.float32)
    ls_ref[...] = jnp.concatenate(
        [jnp.sum(h2 * a2s_ref[...], axis=1, keepdims=True), pad7], axis=1)
    ld_ref[...] = jnp.concatenate(
        [jnp.sum(h2 * a2d_ref[...], axis=1, keepdims=True), pad7], axis=1)
    h2_ref[0] = h2[:, :64]
    h2_ref[1] = h2[:, 64:]


def _mm2(hagg, b1, W2, a2_src, a2_dst):
    BR = 128
    o = hagg.reshape(8, N_PAD, 64)
    return pl.pallas_call(
        _mm2_body,
        grid=(N_PAD // BR,),
        in_specs=[pl.BlockSpec((BR, 64), lambda i: (i, 0))] * 8 + [
            pl.BlockSpec((1, HEADS * HID), lambda i: (0, 0)),
            pl.BlockSpec((HEADS * HID, D_OUT), lambda i: (0, 0)),
            pl.BlockSpec((1, D_OUT), lambda i: (0, 0)),
            pl.BlockSpec((1, D_OUT), lambda i: (0, 0)),
        ],
        out_specs=[
            pl.BlockSpec((2, BR, 64), lambda i: (0, i, 0)),
            pl.BlockSpec((BR, HEADS), lambda i: (i, 0)),
            pl.BlockSpec((BR, HEADS), lambda i: (i, 0)),
        ],
        out_shape=[
            jax.ShapeDtypeStruct((2, N_PAD, 64), jnp.float32),
            jax.ShapeDtypeStruct((N_PAD, HEADS), jnp.float32),
            jax.ShapeDtypeStruct((N_PAD, HEADS), jnp.float32),
        ],
    )(o[0], o[1], o[2], o[3], o[4], o[5], o[6], o[7], b1.reshape(1, -1), W2,
      a2_src.reshape(1, -1), a2_dst.reshape(1, -1))


def _out_body(q0, q1, b2_ref, out_ref):
    out_ref[...] = jnp.concatenate([q0[...], q1[...]], axis=1) + b2_ref[...]


def _combine_out(o2agg, b2):
    BR = 128
    q = o2agg.reshape(2, N_PAD, 64)
    return pl.pallas_call(
        _out_body,
        grid=(N_PAD // BR,),
        in_specs=[
            pl.BlockSpec((BR, 64), lambda i: (i, 0)),
            pl.BlockSpec((BR, 64), lambda i: (i, 0)),
            pl.BlockSpec((1, D_OUT), lambda i: (0, 0)),
        ],
        out_specs=pl.BlockSpec((BR, D_OUT), lambda i: (i, 0)),
        out_shape=jax.ShapeDtypeStruct((N_PAD, D_OUT), jnp.float32),
    )(q[0], q[1], b2.reshape(1, -1))


def _iota16():
    return lax.iota(jnp.int32, 16)


def _make_pass_a(H):
    EPT = E_PAD // (NC * NS)
    NBLK = EPT // KBLK
    NSL = KBLK * H // LANES

    scratch = [
        pltpu.VMEM((KBLK,), jnp.int32),
        pltpu.VMEM((SUB, 128), jnp.int32),
        pltpu.VMEM((KBLK, H), jnp.float32),
        pltpu.VMEM((KBLK, H), jnp.float32),
        pltpu.VMEM((KBLK, H), jnp.float32),
        pltpu.VMEM((H, KBLK), jnp.float32),
        pltpu.VMEM_SHARED((N_PAD, H), jnp.float32),
        pltpu.SemaphoreType.DMA,
    ]

    @functools.partial(
        pl.kernel, mesh=_mesh, compiler_params=_sc_params,
        out_type=(
            jax.ShapeDtypeStruct((2 * N_PAD, H), jnp.float32),
            jax.ShapeDtypeStruct((H * E_PAD,), jnp.float32),
        ),
        scratch_types=scratch,
    )
    def pass_a(src_hbm, dst2d_hbm, asrc_hbm, adst_hbm, zeros_hbm,
               den_out, ext_out,
               src_v, dst_v, asr_v, adr_v, exr_v, ext_v, den_sh, sem):
        c = lax.axis_index("c")
        s = lax.axis_index("s")
        wid = s * NC + c
        base_edges = wid * EPT

        pltpu.sync_copy(zeros_hbm.at[pl.ds(s * ROWS_PER_TILE, ROWS_PER_TILE)],
                        den_sh.at[pl.ds(s * ROWS_PER_TILE, ROWS_PER_TILE)])
        plsc.subcore_barrier()

        def blk(b, carry):
            ebase = pl.multiple_of(base_edges + b * KBLK, KBLK)
            rbase = pl.multiple_of(ebase // 128, SUB)
            pltpu.sync_copy(src_hbm.at[pl.ds(ebase, KBLK)], src_v)
            pltpu.sync_copy(dst2d_hbm.at[pl.ds(rbase, SUB)], dst_v)
            cps = []
            for j in range(SUB):
                cps.append(pltpu.async_copy(
                    asrc_hbm.at[src_v.at[pl.ds(j * 128, 128)]],
                    asr_v.at[pl.ds(j * 128, 128)], sem))
                cps.append(pltpu.async_copy(
                    adst_hbm.at[dst_v.at[j]],
                    adr_v.at[pl.ds(j * 128, 128)], sem))
            for cp in cps:
                cp.wait()

            SH = H.bit_length() - 1

            def comp(v, c2):
                flat = v * LANES + _iota16()
                r = lax.shift_right_logical(flat, SH)
                col = lax.bitwise_and(flat, H - 1)
                e = (plsc.load_gather(asr_v, [r, col])
                     + plsc.load_gather(adr_v, [r, col]))
                e = jnp.where(e >= 0.0, e, 0.2 * e)
                ex = jnp.exp(e)
                plsc.store_scatter(exr_v, [r, col], ex)
                plsc.store_scatter(ext_v, [col, r], ex)
                return c2
            lax.fori_loop(0, NSL, comp, 0, unroll=4)

            for j in range(SUB):
                pltpu.sync_copy(exr_v.at[pl.ds(j * 128, 128)],
                                den_sh.at[dst_v.at[j]], add=True)
            for h in range(H):
                pltpu.sync_copy(
                    ext_v.at[h],
                    ext_out.at[pl.ds(
                        pl.multiple_of(h * E_PAD + ebase, KBLK), KBLK)])
            return carry
        lax.fori_loop(0, NBLK, blk, 0)

        plsc.subcore_barrier()
        pltpu.sync_copy(
            den_sh.at[pl.ds(s * ROWS_PER_TILE, ROWS_PER_TILE)],
            den_out.at[pl.ds(
                pl.multiple_of(c * N_PAD + s * ROWS_PER_TILE, ROWS_PER_TILE),
                ROWS_PER_TILE)])

    return pass_a


def _make_pass_b(NCHUNK, DC, HPC, NHEADS):
    EPT = E_PAD // NS
    NBLK = EPT // KBLK
    CPS = NCHUNK // NC
    DPH = DC // HPC

    scratch = [
        pltpu.VMEM((KBLK,), jnp.int32),
        pltpu.VMEM((SUB, 128), jnp.int32),
        pltpu.VMEM((GROW, DC), jnp.float32),
        pltpu.VMEM((GROW, DC), jnp.float32),
        pltpu.VMEM((GROW // 128, 128), jnp.int32),
    ] + [pltpu.VMEM((KBLK,), jnp.float32) for _ in range(HPC)] \
      + [pltpu.VMEM((N_PAD,), jnp.float32) for _ in range(HPC)] + [
        pltpu.VMEM_SHARED((N_PAD, DC), jnp.float32),
        pltpu.SemaphoreType.DMA,
        pltpu.SemaphoreType.DMA,
        pltpu.SemaphoreType.DMA,
    ]

    @functools.partial(
        pl.kernel, mesh=_mesh, compiler_params=_sc_params,
        out_type=jax.ShapeDtypeStruct((NCHUNK * N_PAD, DC), jnp.float32),
        scratch_types=scratch,
    )
    def pass_b(src_hbm, dst2d_hbm, hblk_hbm, ext_hbm, rdt_hbm,
               zeros_hbm, out_hbm, *refs):
        idx_v = refs[0]
        dst_v = refs[1]
        rows = [refs[2], refs[3]]
        dstS = refs[4]
        ex_vs = [refs[5 + k] for k in range(HPC)]
        rd_vs = [refs[5 + HPC + k] for k in range(HPC)]
        acc_sh = refs[5 + 2 * HPC]
        sem_g = refs[6 + 2 * HPC]
        sem_s = [refs[7 + 2 * HPC], refs[8 + 2 * HPC]]

        c = lax.axis_index("c")
        s = lax.axis_index("s")

        def chunk_body(jj, carry0):
            chunk = c * CPS + jj
            heads = [(chunk * NHEADS) // NCHUNK + hh for hh in range(HPC)]
            for hh in range(HPC):
                pltpu.sync_copy(
                    rdt_hbm.at[pl.ds(
                        pl.multiple_of(heads[hh] * N_PAD, N_PAD), N_PAD)],
                    rd_vs[hh])
            pltpu.sync_copy(
                zeros_hbm.at[pl.ds(s * ROWS_PER_TILE, ROWS_PER_TILE)],
                acc_sh.at[pl.ds(s * ROWS_PER_TILE, ROWS_PER_TILE)])
            plsc.subcore_barrier()

            off = chunk * N_PAD

            NJ = GROW // 128

            def gather_half(half, buf):
                return [pltpu.async_copy(
                    hblk_hbm.at[idx_v.at[pl.ds(half * GROW + j * 128, 128)]],
                    buf.at[pl.ds(j * 128, 128)], sem_g)
                    for j in range(NJ)]

            def drain_tail():
                for j in range(NJ):
                    pltpu.make_async_copy(
                        rows[1].at[pl.ds(j * 128, 128)],
                        acc_sh.at[dstS.at[j]], sem_s[1]).wait()

            def compute_half(half, buf):
                for g in range(GROW // LANES):
                    rowid = g * LANES + _iota16()
                    fo = half * GROW + g * LANES
                    d16 = dst_v.at[fo // 128][pl.ds(fo % 128, LANES)]
                    for hh in range(HPC):
                        exh = ex_vs[hh][pl.ds(fo, LANES)]
                        al = exh * plsc.load_gather(rd_vs[hh], [d16])

                        def col(dd, c2, hh=hh, al=al, rowid=rowid):
                            d = (jnp.zeros((16,), jnp.int32)
                                 + (hh * DPH + dd))
                            cv = plsc.load_gather(buf, [rowid, d])
                            plsc.store_scatter(buf, [rowid, d], cv * al)
                            return c2
                        lax.fori_loop(0, DPH, col, 0, unroll=4)

            def blk(b, carry):
                ebase = pl.multiple_of(s * EPT + b * KBLK, KBLK)
                rbase = pl.multiple_of(ebase // 128, SUB)
                pltpu.sync_copy(src_hbm.at[pl.ds(ebase, KBLK)], idx_v)
                pltpu.sync_copy(dst2d_hbm.at[pl.ds(rbase, SUB)], dst_v)

                def addoff(v, c2):
                    sl = pl.ds(v * LANES, LANES)
                    idx_v[sl] = idx_v[sl] + off
                    return c2
                lax.fori_loop(0, KBLK // LANES, addoff, 0, unroll=4)

                for hh in range(HPC):
                    pltpu.sync_copy(
                        ext_hbm.at[pl.ds(
                            pl.multiple_of(heads[hh] * E_PAD + ebase, KBLK),
                            KBLK)],
                        ex_vs[hh])

                g0 = gather_half(0, rows[0])
                for cp in g0:
                    cp.wait()

                @pl.when(b > 0)
                def _():
                    drain_tail()

                g1 = gather_half(1, rows[1])
                compute_half(0, rows[0])
                s0 = [pltpu.async_copy(
                    rows[0].at[pl.ds(j * 128, 128)],
                    acc_sh.at[dst_v.at[j]], sem_s[0], add=True)
                    for j in range(NJ)]
                for cp in g1:
                    cp.wait()
                compute_half(1, rows[1])
                for cp in s0:
                    cp.wait()
                pltpu.sync_copy(
                    dst2d_hbm.at[pl.ds(pl.multiple_of(rbase + NJ, NJ), NJ)],
                    dstS)
                for j in range(NJ):
                    pltpu.async_copy(
                        rows[1].at[pl.ds(j * 128, 128)],
                        acc_sh.at[dstS.at[j]], sem_s[1], add=True)
                return carry
            lax.fori_loop(0, NBLK, blk, 0)
            drain_tail()

            plsc.subcore_barrier()
            pltpu.sync_copy(
                acc_sh.at[pl.ds(s * ROWS_PER_TILE, ROWS_PER_TILE)],
                out_hbm.at[pl.ds(
                    pl.multiple_of(off + s * ROWS_PER_TILE, ROWS_PER_TILE),
                    ROWS_PER_TILE)])
            plsc.subcore_barrier()
            return carry0
        lax.fori_loop(0, CPS, chunk_body, 0)

    return pass_b


_pass_a_l1 = _make_pass_a(HEADS)
_pass_b_l1 = _make_pass_b(8, 64, 1, 8)
_pass_b_l2 = _make_pass_b(2, 64, 1, 1)


def kernel(x, edge_index, W1, a1_src, a1_dst, b1, W2, a2_src, a2_dst, b2):
    f32 = jnp.float32
    x_pad = jnp.pad(x, ((0, N_PAD - N), (0, 0)))
    src = edge_index[0]
    dst = edge_index[1]
    src_p = jnp.concatenate([src, jnp.zeros((E_PAD - E,), jnp.int32)])
    dst_p = jnp.concatenate([dst, jnp.full((E_PAD - E,), N_PAD - 1, jnp.int32)])
    dst2d = dst_p.reshape(E_PAD // 128, 128)

    eye = jnp.eye(HEADS, dtype=f32)
    Asrc = (a1_src[:, :, None] * eye[:, None, :]).reshape(HEADS * HID, HEADS)
    Adst = (a1_dst[:, :, None] * eye[:, None, :]).reshape(HEADS * HID, HEADS)

    z64 = jnp.zeros((N_PAD, 64), f32)
    z8 = jnp.zeros((N_PAD, HEADS), f32)

    hblk, asrc1, adst1 = _mm1(x_pad, W1, Asrc, Adst)
    den1, ext1 = _pass_a_l1(src_p, dst2d, asrc1, adst1, z8)
    rdt1 = _rdenom(den1, HEADS).reshape(-1)
    hagg = _pass_b_l1(src_p, dst2d, hblk.reshape(8 * N_PAD, 64), ext1,
                      rdt1, z64)

    h2blk, asrc2, adst2 = _mm2(hagg, b1, W2, a2_src, a2_dst)
    den2, ext2 = _pass_a_l1(src_p, dst2d, asrc2, adst2, z8)
    rdt2 = _rdenom(den2, HEADS).reshape(-1)
    o2agg = _pass_b_l2(src_p, dst2d, h2blk.reshape(2 * N_PAD, 64), ext2,
                       rdt2, z64)

    out = _combine_out(o2agg, b2)
    return out[:N]


# speedup vs baseline: 5.3464x; 1.0308x over previous
import functools

import jax
import jax.numpy as jnp
from jax import lax
from jax.experimental import pallas as pl
from jax.experimental.pallas import tpu as pltpu
from jax.experimental.pallas import tpu_sc as plsc

N = 10000
E = 320000
D_IN = 128
HID = 64
HEADS = 8
D_OUT = 128

N_PAD = 10240
E_PAD = 327680
NC, NS, LANES = 2, 16, 16
KBLK = 1024
SUB = KBLK // 128
QROW = 256
ROWS_PER_TILE = N_PAD // NS

_mesh = plsc.VectorSubcoreMesh(
    core_axis_name="c", subcore_axis_name="s", num_cores=NC, num_subcores=NS)
_sc_params = pltpu.CompilerParams(
    needs_layout_passes=False, use_tc_tiling_on_sc=False)


def _mm1_body(x_ref, w_ref, asr_ref, adr_ref, h_ref, s_ref, d_ref):
    h = jnp.dot(x_ref[...], w_ref[...], preferred_element_type=jnp.float32)
    s_ref[...] = jnp.dot(h, asr_ref[...], preferred_element_type=jnp.float32)
    d_ref[...] = jnp.dot(h, adr_ref[...], preferred_element_type=jnp.float32)
    for j in range(8):
        h_ref[j] = h[:, j * 64:(j + 1) * 64]


def _mm1(x_pad, W1, Asrc, Adst):
    BR = 128
    return pl.pallas_call(
        _mm1_body,
        grid=(N_PAD // BR,),
        in_specs=[
            pl.BlockSpec((BR, D_IN), lambda i: (i, 0)),
            pl.BlockSpec((D_IN, HEADS * HID), lambda i: (0, 0)),
            pl.BlockSpec((HEADS * HID, HEADS), lambda i: (0, 0)),
            pl.BlockSpec((HEADS * HID, HEADS), lambda i: (0, 0)),
        ],
        out_specs=[
            pl.BlockSpec((8, BR, 64), lambda i: (0, i, 0)),
            pl.BlockSpec((BR, HEADS), lambda i: (i, 0)),
            pl.BlockSpec((BR, HEADS), lambda i: (i, 0)),
        ],
        out_shape=[
            jax.ShapeDtypeStruct((8, N_PAD, 64), jnp.float32),
            jax.ShapeDtypeStruct((N_PAD, HEADS), jnp.float32),
            jax.ShapeDtypeStruct((N_PAD, HEADS), jnp.float32),
        ],
    )(x_pad, W1, Asrc, Adst)


def _rd_body(p0_ref, p1_ref, out_ref):
    den = p0_ref[...] + p1_ref[...]
    out_ref[...] = jnp.transpose(1.0 / (den + 1e-16))


def _rdenom(partials, H):
    BR = 128
    p = partials.reshape(2, N_PAD, H)
    return pl.pallas_call(
        _rd_body,
        grid=(N_PAD // BR,),
        in_specs=[
            pl.BlockSpec((BR, H), lambda i: (i, 0)),
            pl.BlockSpec((BR, H), lambda i: (i, 0)),
        ],
        out_specs=pl.BlockSpec((H, BR), lambda i: (0, i)),
        out_shape=jax.ShapeDtypeStruct((H, N_PAD), jnp.float32),
    )(p[0], p[1])


def _mm2_body(o0, o1, o2, o3, o4, o5, o6, o7, b1_ref, w2_ref, a2s_ref,
              a2d_ref, h2_ref, ls_ref, ld_ref):
    i = pl.program_id(0)
    hcat = jnp.concatenate(
        [o0[...], o1[...], o2[...], o3[...],
         o4[...], o5[...], o6[...], o7[...]], axis=1)
    h = jnp.maximum(hcat + b1_ref[...], 0.0)
    rows = i * h.shape[0] + lax.broadcasted_iota(jnp.int32, (h.shape[0], 1), 0)
    h = jnp.where(rows < N, h, 0.0)
    h2 = jnp.dot(h, w2_ref[...], preferred_element_type=jnp.float32)
    pad7 = jnp.zeros((h2.shape[0], 7), jnp.float32)
    ls_ref[...] = jnp.concatenate(
        [jnp.sum(h2 * a2s_ref[...], axis=1, keepdims=True), pad7], axis=1)
    ld_ref[...] = jnp.concatenate(
        [jnp.sum(h2 * a2d_ref[...], axis=1, keepdims=True), pad7], axis=1)
    h2_ref[0] = h2[:, :64]
    h2_ref[1] = h2[:, 64:]


def _mm2(hagg, b1, W2, a2_src, a2_dst):
    BR = 128
    o = hagg.reshape(8, N_PAD, 64)
    return pl.pallas_call(
        _mm2_body,
        grid=(N_PAD // BR,),
        in_specs=[pl.BlockSpec((BR, 64), lambda i: (i, 0))] * 8 + [
            pl.BlockSpec((1, HEADS * HID), lambda i: (0, 0)),
            pl.BlockSpec((HEADS * HID, D_OUT), lambda i: (0, 0)),
            pl.BlockSpec((1, D_OUT), lambda i: (0, 0)),
            pl.BlockSpec((1, D_OUT), lambda i: (0, 0)),
        ],
        out_specs=[
            pl.BlockSpec((2, BR, 64), lambda i: (0, i, 0)),
            pl.BlockSpec((BR, HEADS), lambda i: (i, 0)),
            pl.BlockSpec((BR, HEADS), lambda i: (i, 0)),
        ],
        out_shape=[
            jax.ShapeDtypeStruct((2, N_PAD, 64), jnp.float32),
            jax.ShapeDtypeStruct((N_PAD, HEADS), jnp.float32),
            jax.ShapeDtypeStruct((N_PAD, HEADS), jnp.float32),
        ],
    )(o[0], o[1], o[2], o[3], o[4], o[5], o[6], o[7], b1.reshape(1, -1), W2,
      a2_src.reshape(1, -1), a2_dst.reshape(1, -1))


def _out_body(q0, q1, b2_ref, out_ref):
    out_ref[...] = jnp.concatenate([q0[...], q1[...]], axis=1) + b2_ref[...]


def _combine_out(o2agg, b2):
    BR = 128
    q = o2agg.reshape(2, N_PAD, 64)
    return pl.pallas_call(
        _out_body,
        grid=(N_PAD // BR,),
        in_specs=[
            pl.BlockSpec((BR, 64), lambda i: (i, 0)),
            pl.BlockSpec((BR, 64), lambda i: (i, 0)),
            pl.BlockSpec((1, D_OUT), lambda i: (0, 0)),
        ],
        out_specs=pl.BlockSpec((BR, D_OUT), lambda i: (i, 0)),
        out_shape=jax.ShapeDtypeStruct((N_PAD, D_OUT), jnp.float32),
    )(q[0], q[1], b2.reshape(1, -1))


def _iota16():
    return lax.iota(jnp.int32, 16)


def _make_pass_a(H):
    EPT = E_PAD // (NC * NS)
    NBLK = EPT // KBLK
    NSL = KBLK * H // LANES

    scratch = [
        pltpu.VMEM((KBLK,), jnp.int32),
        pltpu.VMEM((SUB, 128), jnp.int32),
        pltpu.VMEM((KBLK, H), jnp.float32),
        pltpu.VMEM((KBLK, H), jnp.float32),
        pltpu.VMEM((KBLK, H), jnp.float32),
        pltpu.VMEM((H, KBLK), jnp.float32),
        pltpu.VMEM_SHARED((N_PAD, H), jnp.float32),
        pltpu.SemaphoreType.DMA,
    ]

    @functools.partial(
        pl.kernel, mesh=_mesh, compiler_params=_sc_params,
        out_type=(
            jax.ShapeDtypeStruct((2 * N_PAD, H), jnp.float32),
            jax.ShapeDtypeStruct((H * E_PAD,), jnp.float32),
        ),
        scratch_types=scratch,
    )
    def pass_a(src_hbm, dst2d_hbm, asrc_hbm, adst_hbm, zeros_hbm,
               den_out, ext_out,
               src_v, dst_v, asr_v, adr_v, exr_v, ext_v, den_sh, sem):
        c = lax.axis_index("c")
        s = lax.axis_index("s")
        wid = s * NC + c
        base_edges = wid * EPT

        pltpu.sync_copy(zeros_hbm.at[pl.ds(s * ROWS_PER_TILE, ROWS_PER_TILE)],
                        den_sh.at[pl.ds(s * ROWS_PER_TILE, ROWS_PER_TILE)])
        plsc.subcore_barrier()

        def blk(b, carry):
            ebase = pl.multiple_of(base_edges + b * KBLK, KBLK)
            rbase = pl.multiple_of(ebase // 128, SUB)
            pltpu.sync_copy(src_hbm.at[pl.ds(ebase, KBLK)], src_v)
            pltpu.sync_copy(dst2d_hbm.at[pl.ds(rbase, SUB)], dst_v)
            cps = []
            for j in range(SUB):
                cps.append(pltpu.async_copy(
                    asrc_hbm.at[src_v.at[pl.ds(j * 128, 128)]],
                    asr_v.at[pl.ds(j * 128, 128)], sem))
                cps.append(pltpu.async_copy(
                    adst_hbm.at[dst_v.at[j]],
                    adr_v.at[pl.ds(j * 128, 128)], sem))
            for cp in cps:
                cp.wait()

            SH = H.bit_length() - 1

            def comp(v, c2):
                flat = v * LANES + _iota16()
                r = lax.shift_right_logical(flat, SH)
                col = lax.bitwise_and(flat, H - 1)
                e = (plsc.load_gather(asr_v, [r, col])
                     + plsc.load_gather(adr_v, [r, col]))
                e = jnp.where(e >= 0.0, e, 0.2 * e)
                ex = jnp.exp(e)
                plsc.store_scatter(exr_v, [r, col], ex)
                plsc.store_scatter(ext_v, [col, r], ex)
                return c2
            lax.fori_loop(0, NSL, comp, 0, unroll=4)

            for j in range(SUB):
                pltpu.sync_copy(exr_v.at[pl.ds(j * 128, 128)],
                                den_sh.at[dst_v.at[j]], add=True)
            for h in range(H):
                pltpu.sync_copy(
                    ext_v.at[h],
                    ext_out.at[pl.ds(
                        pl.multiple_of(h * E_PAD + ebase, KBLK), KBLK)])
            return carry
        lax.fori_loop(0, NBLK, blk, 0)

        plsc.subcore_barrier()
        pltpu.sync_copy(
            den_sh.at[pl.ds(s * ROWS_PER_TILE, ROWS_PER_TILE)],
            den_out.at[pl.ds(
                pl.multiple_of(c * N_PAD + s * ROWS_PER_TILE, ROWS_PER_TILE),
                ROWS_PER_TILE)])

    return pass_a


def _make_pass_b(NCHUNK, DC, HPC, NHEADS):
    EPT = E_PAD // NS
    NBLK = EPT // KBLK
    CPS = NCHUNK // NC
    DPH = DC // HPC

    scratch = [
        pltpu.VMEM((KBLK,), jnp.int32),
        pltpu.VMEM((SUB, 128), jnp.int32),
        pltpu.VMEM((QROW, DC), jnp.float32),
        pltpu.VMEM((QROW, DC), jnp.float32),
        pltpu.VMEM((QROW, DC), jnp.float32),
        pltpu.VMEM((QROW, DC), jnp.float32),
    ] + [pltpu.VMEM((KBLK,), jnp.float32) for _ in range(HPC)] \
      + [pltpu.VMEM((N_PAD,), jnp.float32) for _ in range(HPC)] + [
        pltpu.VMEM_SHARED((N_PAD, DC), jnp.float32),
        pltpu.SemaphoreType.DMA,
        pltpu.SemaphoreType.DMA,
        pltpu.SemaphoreType.DMA,
    ]

    @functools.partial(
        pl.kernel, mesh=_mesh, compiler_params=_sc_params,
        out_type=jax.ShapeDtypeStruct((NCHUNK * N_PAD, DC), jnp.float32),
        scratch_types=scratch,
    )
    def pass_b(src_hbm, dst2d_hbm, hblk_hbm, ext_hbm, rdt_hbm,
               zeros_hbm, out_hbm, *refs):
        idx_v = refs[0]
        dst_v = refs[1]
        rin = [refs[2], refs[3]]
        rout = [refs[4], refs[5]]
        ex_vs = [refs[6 + k] for k in range(HPC)]
        rd_vs = [refs[6 + HPC + k] for k in range(HPC)]
        acc_sh = refs[6 + 2 * HPC]
        sem_g = refs[7 + 2 * HPC]
        sem_s = [refs[8 + 2 * HPC], refs[9 + 2 * HPC]]

        c = lax.axis_index("c")
        s = lax.axis_index("s")

        def chunk_body(jj, carry0):
            chunk = c * CPS + jj
            heads = [(chunk * NHEADS) // NCHUNK + hh for hh in range(HPC)]
            for hh in range(HPC):
                pltpu.sync_copy(
                    rdt_hbm.at[pl.ds(
                        pl.multiple_of(heads[hh] * N_PAD, N_PAD), N_PAD)],
                    rd_vs[hh])
            pltpu.sync_copy(
                zeros_hbm.at[pl.ds(s * ROWS_PER_TILE, ROWS_PER_TILE)],
                acc_sh.at[pl.ds(s * ROWS_PER_TILE, ROWS_PER_TILE)])
            plsc.subcore_barrier()

            off = chunk * N_PAD

            NJ = QROW // 128
            NQ = KBLK // QROW

            def gather_q(q, buf):
                return [pltpu.async_copy(
                    hblk_hbm.at[idx_v.at[pl.ds(q * QROW + j * 128, 128)]],
                    buf.at[pl.ds(j * 128, 128)], sem_g)
                    for j in range(NJ)]

            def scatter_q(q, buf, p):
                for j in range(NJ):
                    pltpu.async_copy(
                        buf.at[pl.ds(j * 128, 128)],
                        acc_sh.at[dst_v.at[q * NJ + j]], sem_s[p], add=True)

            def drain_q(q, buf, p):
                for j in range(NJ):
                    pltpu.make_async_copy(
                        buf.at[pl.ds(j * 128, 128)],
                        acc_sh.at[dst_v.at[q * NJ + j]], sem_s[p]).wait()

            def compute_q(q, bin_, bout):
                for g in range(QROW // LANES):
                    rowid = g * LANES + _iota16()
                    fo = q * QROW + g * LANES
                    d16 = dst_v.at[fo // 128][pl.ds(fo % 128, LANES)]
                    for hh in range(HPC):
                        exh = ex_vs[hh][pl.ds(fo, LANES)]
                        al = exh * plsc.load_gather(rd_vs[hh], [d16])

                        def col(dd, c2, hh=hh, al=al, rowid=rowid):
                            d = (jnp.zeros((16,), jnp.int32)
                                 + (hh * DPH + dd))
                            cv = plsc.load_gather(bin_, [rowid, d])
                            plsc.store_scatter(bout, [rowid, d], cv * al)
                            return c2
                        lax.fori_loop(0, DPH, col, 0, unroll=4)

            def blk(b, carry):
                @pl.when(b > 0)
                def _():
                    drain_q(NQ - 2, rout[0], 0)
                    drain_q(NQ - 1, rout[1], 1)

                ebase = pl.multiple_of(s * EPT + b * KBLK, KBLK)
                rbase = pl.multiple_of(ebase // 128, SUB)
                pltpu.sync_copy(src_hbm.at[pl.ds(ebase, KBLK)], idx_v)
                pltpu.sync_copy(dst2d_hbm.at[pl.ds(rbase, SUB)], dst_v)

                def addoff(v, c2):
                    sl = pl.ds(v * LANES, LANES)
                    idx_v[sl] = idx_v[sl] + off
                    return c2
                lax.fori_loop(0, KBLK // LANES, addoff, 0, unroll=4)

                for hh in range(HPC):
                    pltpu.sync_copy(
                        ext_hbm.at[pl.ds(
                            pl.multiple_of(heads[hh] * E_PAD + ebase, KBLK),
                            KBLK)],
                        ex_vs[hh])

                gq = gather_q(0, rin[0])
                for q in range(NQ):
                    p = q % 2
                    for cp in gq:
                        cp.wait()
                    if q + 1 < NQ:
                        gq = gather_q(q + 1, rin[1 - p])
                    if q >= 2:
                        drain_q(q - 2, rout[p], p)
                    compute_q(q, rin[p], rout[p])
                    scatter_q(q, rout[p], p)
                return carry
            lax.fori_loop(0, NBLK, blk, 0)
            drain_q(NQ - 2, rout[0], 0)
            drain_q(NQ - 1, rout[1], 1)

            plsc.subcore_barrier()
            pltpu.sync_copy(
                acc_sh.at[pl.ds(s * ROWS_PER_TILE, ROWS_PER_TILE)],
                out_hbm.at[pl.ds(
                    pl.multiple_of(off + s * ROWS_PER_TILE, ROWS_PER_TILE),
                    ROWS_PER_TILE)])
            plsc.subcore_barrier()
            return carry0
        lax.fori_loop(0, CPS, chunk_body, 0)

    return pass_b


_pass_a_l1 = _make_pass_a(HEADS)
_pass_b_l1 = _make_pass_b(8, 64, 1, 8)
_pass_b_l2 = _make_pass_b(2, 64, 1, 1)


def kernel(x, edge_index, W1, a1_src, a1_dst, b1, W2, a2_src, a2_dst, b2):
    f32 = jnp.float32
    x_pad = jnp.pad(x, ((0, N_PAD - N), (0, 0)))
    src = edge_index[0]
    dst = edge_index[1]
    src_p = jnp.concatenate([src, jnp.zeros((E_PAD - E,), jnp.int32)])
    dst_p = jnp.concatenate([dst, jnp.full((E_PAD - E,), N_PAD - 1, jnp.int32)])
    dst2d = dst_p.reshape(E_PAD // 128, 128)

    eye = jnp.eye(HEADS, dtype=f32)
    Asrc = (a1_src[:, :, None] * eye[:, None, :]).reshape(HEADS * HID, HEADS)
    Adst = (a1_dst[:, :, None] * eye[:, None, :]).reshape(HEADS * HID, HEADS)

    z64 = jnp.zeros((N_PAD, 64), f32)
    z8 = jnp.zeros((N_PAD, HEADS), f32)

    hblk, asrc1, adst1 = _mm1(x_pad, W1, Asrc, Adst)
    den1, ext1 = _pass_a_l1(src_p, dst2d, asrc1, adst1, z8)
    rdt1 = _rdenom(den1, HEADS).reshape(-1)
    hagg = _pass_b_l1(src_p, dst2d, hblk.reshape(8 * N_PAD, 64), ext1,
                      rdt1, z64)

    h2blk, asrc2, adst2 = _mm2(hagg, b1, W2, a2_src, a2_dst)
    den2, ext2 = _pass_a_l1(src_p, dst2d, asrc2, adst2, z8)
    rdt2 = _rdenom(den2, HEADS).reshape(-1)
    o2agg = _pass_b_l2(src_p, dst2d, h2blk.reshape(2 * N_PAD, 64), ext2,
                       rdt2, z64)

    out = _combine_out(o2agg, b2)
    return out[:N]
